# baseline (speedup 1.0000x reference)
import functools
import jax
import jax.numpy as jnp
from jax import lax
from jax.experimental import pallas as pl
from jax.experimental.pallas import tpu as pltpu

XY_GRID = (432, 496)
XY_R = ((0.0, 69.12), (-39.68, 39.68), (-3.0, 1.0))
CYL_GRID = (2560, 100)
CYL_R = ((-3.141592653589793, 3.141592653589793), (-3.0, 1.0), (0.0, 69.12))
NPTS = 30000
B = 2

NP_PAD = 30720
BN_ROWS = 3072
N_BLOCKS = NP_PAD // BN_ROWS


def _pad_pts(x, value=0.0):
    pad = [(0, 0)] * x.ndim
    pad[1] = (0, NP_PAD - x.shape[1])
    return jnp.pad(x, pad, constant_values=value)


def _pad_to(x, n, axis, value=0.0):
    pad = [(0, 0)] * x.ndim
    pad[axis] = (0, n - x.shape[axis])
    return jnp.pad(x, pad, constant_values=value)


def _mm_stats_body(x_ref, wt_ref, h_ref, stats_ref, acc_ref):
    b = pl.program_id(0)
    j = pl.program_id(1)

    @pl.when(jnp.logical_and(b == 0, j == 0))
    def _():
        acc_ref[...] = jnp.zeros_like(acc_ref)

    h = jnp.dot(x_ref[0], wt_ref[...], preferred_element_type=jnp.float32)
    h_ref[0] = h
    acc_ref[0, :] += jnp.sum(h, axis=0)
    acc_ref[1, :] += jnp.sum(h * h, axis=0)

    @pl.when(jnp.logical_and(b == B - 1, j == N_BLOCKS - 1))
    def _():
        stats_ref[...] = acc_ref[...]


def _pn_matmul_stats(x, wt):
    cp = x.shape[-1]
    return pl.pallas_call(
        _mm_stats_body,
        grid=(B, N_BLOCKS),
        in_specs=[
            pl.BlockSpec((1, BN_ROWS, cp), lambda b, j: (b, j, 0)),
            pl.BlockSpec((cp, 64), lambda b, j: (0, 0)),
        ],
        out_specs=[
            pl.BlockSpec((1, BN_ROWS, 64), lambda b, j: (b, j, 0)),
            pl.BlockSpec((2, 64), lambda b, j: (0, 0)),
        ],
        out_shape=[
            jax.ShapeDtypeStruct((B, NP_PAD, 64), jnp.float32),
            jax.ShapeDtypeStruct((2, 64), jnp.float32),
        ],
        scratch_shapes=[pltpu.VMEM((2, 64), jnp.float32)],
    )(x, wt)


def _norm_body(h_ref, stats_ref, gb_ref, mask_ref, h2_ref, papre_ref, ca_ref,
               ca_acc):
    b = pl.program_id(0)
    j = pl.program_id(1)
    cnt = float(B * NPTS)
    mu = stats_ref[0, :] / cnt
    var = stats_ref[1, :] / cnt - mu * mu
    gamma = gb_ref[0, :]
    beta = gb_ref[1, :]
    h = h_ref[0]
    h2 = (h - mu[None, :]) * jax.lax.rsqrt(var + 1e-3)[None, :]
    h2 = h2 * gamma[None, :] + beta[None, :]
    h2 = jnp.maximum(h2, 0.0) * mask_ref[0, 0, 0][:, None]
    h2_ref[0] = h2
    papre_ref[0, 0, 0, :] = jnp.max(h2, axis=1)
    blk_ca = jnp.max(h2, axis=0, keepdims=True)

    @pl.when(j == 0)
    def _():
        ca_acc[pl.ds(b, 1), :] = blk_ca

    @pl.when(j != 0)
    def _():
        ca_acc[pl.ds(b, 1), :] = jnp.maximum(ca_acc[pl.ds(b, 1), :], blk_ca)

    @pl.when(jnp.logical_and(b == B - 1, j == N_BLOCKS - 1))
    def _():
        ca_ref[...] = ca_acc[...]


def _pn_normalize(h, stats, gamma, beta, mask):
    gb = jnp.stack([gamma, beta], axis=0)
    mask4 = mask.reshape(B, N_BLOCKS, 1, BN_ROWS)
    h2, papre4, ca = pl.pallas_call(
        _norm_body,
        grid=(B, N_BLOCKS),
        in_specs=[
            pl.BlockSpec((1, BN_ROWS, 64), lambda b, j: (b, j, 0)),
            pl.BlockSpec((2, 64), lambda b, j: (0, 0)),
            pl.BlockSpec((2, 64), lambda b, j: (0, 0)),
            pl.BlockSpec((1, 1, 1, BN_ROWS), lambda b, j: (b, j, 0, 0)),
        ],
        out_specs=[
            pl.BlockSpec((1, BN_ROWS, 64), lambda b, j: (b, j, 0)),
            pl.BlockSpec((1, 1, 1, BN_ROWS), lambda b, j: (b, j, 0, 0)),
            pl.BlockSpec((B, 64), lambda b, j: (0, 0)),
        ],
        out_shape=[
            jax.ShapeDtypeStruct((B, NP_PAD, 64), jnp.float32),
            jax.ShapeDtypeStruct((B, N_BLOCKS, 1, BN_ROWS), jnp.float32),
            jax.ShapeDtypeStruct((B, 64), jnp.float32),
        ],
        scratch_shapes=[pltpu.VMEM((B, 64), jnp.float32)],
    )(h, stats, gb, mask4)
    return h2, papre4.reshape(B, NP_PAD), ca


def _attn_body(papre_ref, capre_ref, w1t_ref, b1_ref, w2t_ref, b2_ref,
               cw1t_ref, cb1_ref, cw2t_ref, cb2_ref, pa_ref, ca_ref):
    t = jnp.dot(papre_ref[...], w1t_ref[...], preferred_element_type=jnp.float32)
    t = jnp.maximum(t + b1_ref[0, :][None, :], 0.0)
    pa_ref[...] = jnp.dot(t, w2t_ref[...], preferred_element_type=jnp.float32) \
        + b2_ref[0, :][None, :]
    c = jnp.dot(capre_ref[...], cw1t_ref[...], preferred_element_type=jnp.float32)
    c = jnp.maximum(c + cb1_ref[0, :][None, :], 0.0)
    ca_ref[...] = jnp.dot(c, cw2t_ref[...], preferred_element_type=jnp.float32) \
        + cb2_ref[0, :][None, :]


def _pn_attention(papre, capre, p):
    w1t = _pad_to(p['pa_w1'], NP_PAD, 1).T
    w2t = _pad_to(p['pa_w2'], NP_PAD, 0).T
    b1 = p['pa_b1'][None, :]
    b2 = _pad_to(p['pa_b2'], NP_PAD, 0)[None, :]
    cw1t = p['ca_w1'].T
    cw2t = p['ca_w2'].T
    cb1 = p['ca_b1'][None, :]
    cb2 = p['ca_b2'][None, :]
    w1tp = _pad_to(w1t, 8, 1)
    w2tp = _pad_to(w2t, 8, 0)
    b1p = _pad_to(b1, 8, 1)
    cw1tp = _pad_to(cw1t, 8, 1)
    cw2tp = _pad_to(cw2t, 8, 0)
    cb1p = _pad_to(cb1, 8, 1)
    pa, ca = pl.pallas_call(
        _attn_body,
        out_shape=[
            jax.ShapeDtypeStruct((B, NP_PAD), jnp.float32),
            jax.ShapeDtypeStruct((B, 64), jnp.float32),
        ],
    )(papre, capre, w1tp, b1p, w2tp, b2, cw1tp, cb1p, cw2tp, cb2)
    return pa, ca


def _combine_body(h2_ref, pa_ref, ca_ref, out_ref):
    b = pl.program_id(0)
    pa = pa_ref[0, 0, 0]
    ca = ca_ref[pl.ds(b, 1), :]
    w = jax.nn.sigmoid(pa[:, None] * ca)
    out_ref[0] = h2_ref[0] * w


def _pn_combine(h2, pa, ca):
    pa4 = pa.reshape(B, N_BLOCKS, 1, BN_ROWS)
    return pl.pallas_call(
        _combine_body,
        grid=(B, N_BLOCKS),
        in_specs=[
            pl.BlockSpec((1, BN_ROWS, 64), lambda b, j: (b, j, 0)),
            pl.BlockSpec((1, 1, 1, BN_ROWS), lambda b, j: (b, j, 0, 0)),
            pl.BlockSpec((B, 64), lambda b, j: (0, 0)),
        ],
        out_specs=pl.BlockSpec((1, BN_ROWS, 64), lambda b, j: (b, j, 0)),
        out_shape=jax.ShapeDtypeStruct((B, NP_PAD, 64), jnp.float32),
    )(h2, pa4, ca)


def _pointnet(x_padded, mask_padded, p):
    cin = p['W'].shape[1]
    cp = x_padded.shape[-1]
    wt = _pad_to(p['W'].T, cp, 0)
    h, stats = _pn_matmul_stats(x_padded, wt)
    h2, papre, capre = _pn_normalize(h, stats, p['gamma'], p['beta'], mask_padded)
    pa, ca = _pn_attention(papre, capre, p)
    return _pn_combine(h2, pa, ca)


def _segment_sum(data, seg, num_seg):
    b, n, c = data.shape
    offs = (jnp.arange(b, dtype=seg.dtype) * num_seg)[:, None]
    flat = (seg + offs).reshape(-1)
    out = jax.ops.segment_sum(data.reshape(b * n, c), flat, num_segments=b * num_seg)
    return out.reshape(b, num_seg, c)


def _segment_max(data, seg, num_seg, paddings):
    b, n, c = data.shape
    valid = (paddings < 0.5)[..., None]
    d = jnp.where(valid, data, -1e20)
    offs = (jnp.arange(b, dtype=seg.dtype) * num_seg)[:, None]
    flat = (seg + offs).reshape(-1)
    out = jax.ops.segment_max(d.reshape(b * n, c), flat, num_segments=b * num_seg)
    out = out.reshape(b, num_seg, c)
    return jnp.where(out <= -1e19, 0.0, out)


def _gather_rows(table, idx):
    return jax.vmap(lambda t, i: t[i])(table, idx)


def _to_cylinder(p):
    x = p[..., 0]; y = p[..., 1]; z = p[..., 2]
    rho = jnp.sqrt(x * x + y * y)
    theta = jnp.arctan2(y, x)
    return jnp.stack([theta, z, rho], axis=-1)


def _points_to_voxels(p, mask, grid, ranges):
    gx, gy = grid; gz = 1
    (x0, x1), (y0, y1), (z0, z1) = ranges
    vs = jnp.array([(x1 - x0) / gx, (y1 - y0) / gy, (z1 - z0) / gz], jnp.float32)
    off = jnp.array([x0, y0, z0], jnp.float32)
    vxyz = (p - off) / vs
    coords = jnp.floor(vxyz).astype(jnp.int32)
    lim = jnp.array([gx, gy, gz], jnp.int32)
    in_range = jnp.all((coords >= 0) & (coords < lim), axis=-1)
    valid = in_range & (mask > 0.5)
    paddings = 1.0 - valid.astype(jnp.float32)
    cc = jnp.clip(coords, 0, lim - 1)
    idx = cc[..., 0] * (gy * gz) + cc[..., 1] * gz + cc[..., 2]
    centers = (cc.astype(jnp.float32) + 0.5) * vs + off
    nv = gx * gy * gz
    cnt = _segment_sum((1.0 - paddings)[..., None], idx, nv)
    ppc = _gather_rows(cnt[..., 0], idx) * (1.0 - paddings)
    return {'indices': idx, 'paddings': paddings, 'num_voxels': nv,
            'centers': centers, 'voxel_xyz': vxyz, 'voxel_point_count': ppc}


def _voxel_stats(p, vox):
    idx = vox['indices']; nv = vox['num_voxels']
    m = (1.0 - vox['paddings'])[..., None]
    cnt = _segment_sum(m, idx, nv)
    s = _segment_sum(p * m, idx, nv)
    mean = s / jnp.maximum(cnt, 1.0)
    mean_p = _gather_rows(mean, idx)
    centroids = mean_p * m
    centered = (p - mean_p) * m
    outer = (centered[..., :, None] * centered[..., None, :]).reshape(
        p.shape[0], p.shape[1], 9)
    cov = _segment_sum(outer, idx, nv) / jnp.maximum(cnt, 1.0)
    cov_p = _gather_rows(cov, idx) * m
    return {'centered_xyz': centered, 'points_covariance': cov_p,
            'centroids': centroids}


def _bilinear(im, xy):
    b, hh, ww, c = im.shape
    x = xy[..., 0]; y = xy[..., 1]
    x0 = jnp.floor(x).astype(jnp.int32); x1 = x0 + 1
    y0 = jnp.floor(y).astype(jnp.int32); y1 = y0 + 1
    x0c = jnp.clip(x0, 0, ww - 1); x1c = jnp.clip(x1, 0, ww - 1)
    y0c = jnp.clip(y0, 0, hh - 1); y1c = jnp.clip(y1, 0, hh - 1)
    flat = im.reshape(b, hh * ww, c)
    def g(yy, xx):
        return _gather_rows(flat, yy * ww + xx)
    Ia = g(y0c, x0c); Ib = g(y1c, x0c); Ic = g(y0c, x1c); Id = g(y1c, x1c)
    x0f = x0.astype(jnp.float32); x1f = x1.astype(jnp.float32)
    y0f = y0.astype(jnp.float32); y1f = y1.astype(jnp.float32)
    wa = (x1f - x) * (y1f - y); wb = (x1f - x) * (y - y0f)
    wc = (x - x0f) * (y1f - y); wd = (x - x0f) * (y - y0f)
    return (Ia * wa[..., None] + Ib * wb[..., None]
            + Ic * wc[..., None] + Id * wd[..., None])


def _single_view(x_pad, mask_pad, vox, grid, p):
    h_pad = _pointnet(x_pad, mask_pad, p['pn'])
    h = h_pad[:, :NPTS]
    v = _segment_max(h, vox['indices'], vox['num_voxels'], vox['paddings'])
    b = B; gx, gy = grid
    v = v.reshape(b, gx, gy, 64)
    v = v @ p['conv_w'].T
    mu = v.mean(axis=(0, 1, 2)); var = v.var(axis=(0, 1, 2))
    v = (v - mu) / jnp.sqrt(var + 1e-3) * p['bn_g'] + p['bn_b']
    v = jax.nn.relu(v)
    im = jnp.transpose(v, (0, 2, 1, 3))
    return _bilinear(im, vox['voxel_xyz'][:, :, :2])


def kernel(points_xyz, points_feature, points_mask, params):
    b = points_xyz.shape[0]
    xyv = _points_to_voxels(points_xyz, points_mask, XY_GRID, XY_R)
    xys = _voxel_stats(points_xyz, xyv)
    xyp = points_xyz - xyv['centers']
    pc = _to_cylinder(points_xyz)
    cyv = _points_to_voxels(pc, points_mask, CYL_GRID, CYL_R)
    cys = _voxel_stats(pc, cyv)
    cyp = pc - cyv['centers']
    feat = jnp.concatenate(
        [points_xyz, xyp, xyv['voxel_point_count'][..., None],
         xys['centered_xyz'], xys['points_covariance'], xys['centroids'],
         pc, cyp, cyv['voxel_point_count'][..., None],
         cys['centered_xyz'], cys['points_covariance'], cys['centroids'],
         points_feature[..., None]], axis=-1)
    mask_pad = _pad_pts(points_mask)
    feat_pad = _pad_to(_pad_pts(feat), 128, 2)
    x_pad = _pointnet(feat_pad, mask_pad, params['pn1'])
    x_pad128 = _pad_to(x_pad, 128, 2)
    xxy = _single_view(x_pad128, mask_pad, xyv, XY_GRID, params['xy'])
    xcy = _single_view(x_pad128, mask_pad, cyv, CYL_GRID, params['cyl'])
    xpw = _pointnet(x_pad128, mask_pad, params['pn2'])[:, :NPTS]
    x2 = jnp.concatenate([xxy, xcy, xpw], axis=-1)
    x2_pad = _pad_to(_pad_pts(x2), 256, 2)
    x3 = _pointnet(x2_pad, mask_pad, params['pn3'])[:, :NPTS]
    pil = _segment_max(x3, xyv['indices'], xyv['num_voxels'], xyv['paddings'])
    pil = pil.reshape(b, XY_GRID[0], XY_GRID[1], 64).transpose(0, 3, 2, 1)
    return pil
